# baseline (speedup 1.0000x reference)
import functools

import jax
import jax.numpy as jnp
from jax import lax
from jax.experimental import pallas as pl
from jax.experimental.pallas import tpu as pltpu
from jax.experimental.pallas import tpu_sc as plsc

_DIM = 64
_B = 16384 * 26
_NC = 2
_NS = 16
_NW = _NC * _NS
_BPW = _B // _NW
_CH = 512
_NCH = _BPW // _CH

_mesh = plsc.VectorSubcoreMesh(core_axis_name="c", subcore_axis_name="s")


@functools.partial(
    pl.kernel,
    mesh=_mesh,
    out_type=jax.ShapeDtypeStruct((_B, _DIM), jnp.float32),
    scratch_types=[
        pltpu.VMEM((_CH,), jnp.int32),
        pltpu.VMEM((_CH, _DIM), jnp.float32),
        pltpu.SemaphoreType.DMA,
    ],
    compiler_params=pltpu.CompilerParams(use_tc_tiling_on_sc=False),
)
def _gather(table_hbm, idx_hbm, out_hbm, idx_v, rows_v, sem):
    wid = lax.axis_index("s") * _NC + lax.axis_index("c")
    base = wid * _BPW

    def body(c, carry):
        off = base + c * _CH
        pltpu.sync_copy(idx_hbm.at[pl.ds(off, _CH)], idx_v)
        pltpu.async_copy(table_hbm.at[idx_v], rows_v, sem).wait()
        pltpu.sync_copy(rows_v, out_hbm.at[pl.ds(off, _CH)])
        return carry

    lax.fori_loop(0, _NCH, body, 0)


def kernel(idx, table):
    flat = idx.reshape(-1).astype(jnp.int32)
    out = _gather(table, flat)
    return out.reshape(idx.shape[0], idx.shape[1], _DIM)


# speedup vs baseline: 1.0276x; 1.0276x over previous
import functools

import jax
import jax.numpy as jnp
from jax import lax
from jax.experimental import pallas as pl
from jax.experimental.pallas import tpu as pltpu
from jax.experimental.pallas import tpu_sc as plsc

_DIM = 64
_B = 16384 * 26
_NC = 2
_NS = 16
_NW = _NC * _NS
_BPW = _B // _NW
_CH = 832
_NCH = _BPW // _CH

_mesh = plsc.VectorSubcoreMesh(core_axis_name="c", subcore_axis_name="s")


@functools.partial(
    pl.kernel,
    mesh=_mesh,
    out_type=jax.ShapeDtypeStruct((_B, _DIM), jnp.float32),
    scratch_types=[
        pltpu.VMEM((_NCH, _CH), jnp.int32),
        pltpu.VMEM((_CH, _DIM), jnp.float32),
        pltpu.VMEM((_CH, _DIM), jnp.float32),
        pltpu.SemaphoreType.DMA,
        pltpu.SemaphoreType.DMA,
    ],
    compiler_params=pltpu.CompilerParams(use_tc_tiling_on_sc=False),
)
def _gather(table_hbm, idx_hbm, out_hbm, idx_v, rows0, rows1, gsem, osem):
    wid = lax.axis_index("s") * _NC + lax.axis_index("c")
    base = wid * _BPW
    pltpu.sync_copy(idx_hbm.at[pl.ds(wid * _NCH, _NCH)], idx_v)

    rows = (rows0, rows1)
    gathers = [None, None]
    writes = [None, None]
    gathers[0] = pltpu.async_copy(table_hbm.at[idx_v.at[0]], rows0, gsem)
    for c in range(_NCH):
        cur, nxt = c % 2, (c + 1) % 2
        if c + 1 < _NCH:
            if writes[nxt] is not None:
                writes[nxt].wait()
            gathers[nxt] = pltpu.async_copy(
                table_hbm.at[idx_v.at[c + 1]], rows[nxt], gsem)
        gathers[cur].wait()
        writes[cur] = pltpu.async_copy(
            rows[cur], out_hbm.at[pl.ds(base + c * _CH, _CH)], osem)
    writes[(_NCH - 1) % 2].wait()
    writes[_NCH % 2].wait()


def kernel(idx, table):
    flat = idx.reshape(_B // _CH, _CH).astype(jnp.int32)
    out = _gather(table, flat)
    return out.reshape(idx.shape[0], idx.shape[1], _DIM)


# speedup vs baseline: 1.0644x; 1.0358x over previous
import functools

import jax
import jax.numpy as jnp
from jax import lax
from jax.experimental import pallas as pl
from jax.experimental.pallas import tpu as pltpu
from jax.experimental.pallas import tpu_sc as plsc

_DIM = 64
_FIELDS = 26
_BATCH = 16384
_NC = 2
_NS = 16
_NW = _NC * _NS
_CH = _BATCH // _NW

_mesh = plsc.VectorSubcoreMesh(core_axis_name="c", subcore_axis_name="s")


@functools.partial(
    pl.kernel,
    mesh=_mesh,
    out_type=jax.ShapeDtypeStruct((_FIELDS, _BATCH, _DIM), jnp.float32),
    scratch_types=[
        pltpu.VMEM((_CH,), jnp.int32),
        pltpu.VMEM((_CH,), jnp.int32),
        pltpu.VMEM((_CH, _DIM), jnp.float32),
        pltpu.VMEM((_CH, _DIM), jnp.float32),
        pltpu.SemaphoreType.DMA,
        pltpu.SemaphoreType.DMA,
    ],
    compiler_params=pltpu.CompilerParams(use_tc_tiling_on_sc=False),
)
def _gather(table_hbm, idx_hbm, out_hbm, idx0, idx1, rows0, rows1, gsem, osem):
    wid = lax.axis_index("s") * _NC + lax.axis_index("c")
    col = wid * _CH

    idxb = (idx0, idx1)
    rows = (rows0, rows1)
    gathers = [None, None]
    writes = [None, None]

    pltpu.sync_copy(idx_hbm.at[0, pl.ds(col, _CH)], idx0)
    gathers[0] = pltpu.async_copy(table_hbm.at[idx0], rows0, gsem)
    for c in range(_FIELDS):
        cur, nxt = c % 2, (c + 1) % 2
        if c + 1 < _FIELDS:
            if writes[nxt] is not None:
                writes[nxt].wait()
            pltpu.sync_copy(idx_hbm.at[c + 1, pl.ds(col, _CH)], idxb[nxt])
            gathers[nxt] = pltpu.async_copy(
                table_hbm.at[idxb[nxt]], rows[nxt], gsem)
        gathers[cur].wait()
        writes[cur] = pltpu.async_copy(
            rows[cur], out_hbm.at[c, pl.ds(col, _CH)], osem)
    writes[(_FIELDS - 1) % 2].wait()
    writes[_FIELDS % 2].wait()


def kernel(idx, table):
    idx_t = jnp.swapaxes(idx, 0, 1).astype(jnp.int32)
    out_t = _gather(table, idx_t)
    return jnp.swapaxes(out_t, 0, 1)


# speedup vs baseline: 1.0753x; 1.0102x over previous
import functools

import jax
import jax.numpy as jnp
from jax import lax
from jax.experimental import pallas as pl
from jax.experimental.pallas import tpu as pltpu
from jax.experimental.pallas import tpu_sc as plsc

_DIM = 64
_PADW = 128
_FIELDS = 26
_BATCH = 16384
_NC = 2
_NS = 16
_NW = _NC * _NS
_CH = 256
_NCHUNK = _FIELDS * _BATCH // (_NW * _CH)

_mesh = plsc.VectorSubcoreMesh(core_axis_name="c", subcore_axis_name="s")


@functools.partial(
    pl.kernel,
    mesh=_mesh,
    out_type=jax.ShapeDtypeStruct((_FIELDS, _BATCH, _DIM), jnp.float32),
    scratch_types=[
        pltpu.VMEM((_CH,), jnp.int32),
        pltpu.VMEM((_CH,), jnp.int32),
        pltpu.VMEM((_CH, _PADW), jnp.float32),
        pltpu.VMEM((_CH, _PADW), jnp.float32),
        pltpu.SemaphoreType.DMA,
        pltpu.SemaphoreType.DMA,
    ],
    compiler_params=pltpu.CompilerParams(use_tc_tiling_on_sc=False),
)
def _gather(table_hbm, idx_hbm, out_hbm, idx0, idx1, rows0, rows1, gsem, osem):
    wid = lax.axis_index("s") * _NC + lax.axis_index("c")
    col0 = wid * _CH

    idxb = (idx0, idx1)
    rows = (rows0, rows1)
    gathers = [None, None]
    writes = [None, None]

    def chunk(k):
        return k // 2, col0 + (k % 2) * (_NW * _CH)

    f0, c0 = chunk(0)
    pltpu.sync_copy(idx_hbm.at[f0, pl.ds(c0, _CH)], idx0)
    gathers[0] = pltpu.async_copy(table_hbm.at[idx0], rows0, gsem)
    for k in range(_NCHUNK):
        cur, nxt = k % 2, (k + 1) % 2
        if k + 1 < _NCHUNK:
            if writes[nxt] is not None:
                writes[nxt].wait()
            f, c = chunk(k + 1)
            pltpu.sync_copy(idx_hbm.at[f, pl.ds(c, _CH)], idxb[nxt])
            gathers[nxt] = pltpu.async_copy(
                table_hbm.at[idxb[nxt]], rows[nxt], gsem)
        gathers[cur].wait()
        f, c = chunk(k)
        writes[cur] = pltpu.async_copy(
            rows[cur].at[:, pl.ds(0, _DIM)], out_hbm.at[f, pl.ds(c, _CH)], osem)
    writes[(_NCHUNK - 1) % 2].wait()
    writes[_NCHUNK % 2].wait()


def kernel(idx, table):
    idx_t = jnp.swapaxes(idx, 0, 1).astype(jnp.int32)
    t_pad = jnp.pad(table, ((0, 0), (0, _PADW - _DIM)))
    out_t = _gather(t_pad, idx_t)
    return jnp.swapaxes(out_t, 0, 1)


# speedup vs baseline: 1.0828x; 1.0070x over previous
import functools

import jax
import jax.numpy as jnp
from jax import lax
from jax.experimental import pallas as pl
from jax.experimental.pallas import tpu as pltpu
from jax.experimental.pallas import tpu_sc as plsc

_DIM = 64
_PADW = 128
_FIELDS = 26
_BATCH = 16384
_NC = 2
_NS = 16
_NW = _NC * _NS
_CH = 256
_NCHUNK = _FIELDS * _BATCH // (_NW * _CH)

_mesh = plsc.VectorSubcoreMesh(core_axis_name="c", subcore_axis_name="s")


@functools.partial(
    pl.kernel,
    mesh=_mesh,
    out_type=jax.ShapeDtypeStruct((_FIELDS, _BATCH, _PADW), jnp.float32),
    scratch_types=[
        pltpu.VMEM((_CH,), jnp.int32),
        pltpu.VMEM((_CH,), jnp.int32),
        pltpu.VMEM((_CH, _PADW), jnp.float32),
        pltpu.VMEM((_CH, _PADW), jnp.float32),
        pltpu.SemaphoreType.DMA,
        pltpu.SemaphoreType.DMA,
    ],
    compiler_params=pltpu.CompilerParams(use_tc_tiling_on_sc=False),
)
def _gather(table_hbm, idx_hbm, out_hbm, idx0, idx1, rows0, rows1, gsem, osem):
    wid = lax.axis_index("s") * _NC + lax.axis_index("c")
    col0 = wid * _CH

    idxb = (idx0, idx1)
    rows = (rows0, rows1)
    gathers = [None, None]
    writes = [None, None]

    def chunk(k):
        return k // 2, col0 + (k % 2) * (_NW * _CH)

    f0, c0 = chunk(0)
    pltpu.sync_copy(idx_hbm.at[f0, pl.ds(c0, _CH)], idx0)
    gathers[0] = pltpu.async_copy(table_hbm.at[idx0], rows0, gsem)
    for k in range(_NCHUNK):
        cur, nxt = k % 2, (k + 1) % 2
        if k + 1 < _NCHUNK:
            if writes[nxt] is not None:
                writes[nxt].wait()
            f, c = chunk(k + 1)
            pltpu.sync_copy(idx_hbm.at[f, pl.ds(c, _CH)], idxb[nxt])
            gathers[nxt] = pltpu.async_copy(
                table_hbm.at[idxb[nxt]], rows[nxt], gsem)
        gathers[cur].wait()
        f, c = chunk(k)
        writes[cur] = pltpu.async_copy(
            rows[cur], out_hbm.at[f, pl.ds(c, _CH)], osem)
    writes[(_NCHUNK - 1) % 2].wait()
    writes[_NCHUNK % 2].wait()


def kernel(idx, table):
    idx_t = jnp.swapaxes(idx, 0, 1).astype(jnp.int32)
    t_pad = jnp.pad(table, ((0, 0), (0, _PADW - _DIM)))
    out_t = _gather(t_pad, idx_t)
    return jnp.swapaxes(out_t[:, :, :_DIM], 0, 1)


# speedup vs baseline: 1.0867x; 1.0036x over previous
import functools

import jax
import jax.numpy as jnp
from jax import lax
from jax.experimental import pallas as pl
from jax.experimental.pallas import tpu as pltpu
from jax.experimental.pallas import tpu_sc as plsc

_DIM = 64
_PADW = 128
_FIELDS = 26
_BATCH = 16384
_NC = 2
_NS = 16
_NW = _NC * _NS
_CH = 256
_NCHUNK = _FIELDS * _BATCH // (_NW * _CH)
_NB = 3

_mesh = plsc.VectorSubcoreMesh(core_axis_name="c", subcore_axis_name="s")


@functools.partial(
    pl.kernel,
    mesh=_mesh,
    out_type=jax.ShapeDtypeStruct((_FIELDS, _BATCH, _PADW), jnp.float32),
    scratch_types=[
        pltpu.VMEM((_NB, _CH), jnp.int32),
        pltpu.VMEM((_CH, _PADW), jnp.float32),
        pltpu.VMEM((_CH, _PADW), jnp.float32),
        pltpu.VMEM((_CH, _PADW), jnp.float32),
        pltpu.SemaphoreType.DMA,
        pltpu.SemaphoreType.DMA,
        pltpu.SemaphoreType.DMA,
    ],
    compiler_params=pltpu.CompilerParams(use_tc_tiling_on_sc=False),
)
def _gather(table_hbm, idx_hbm, out_hbm, idx_v, r0, r1, r2, isem, gsem, osem):
    wid = lax.axis_index("s") * _NC + lax.axis_index("c")
    col0 = wid * _CH

    rows = (r0, r1, r2)
    icopies = [None] * _NB
    gathers = [None] * _NB
    writes = [None] * _NB

    def chunk(k):
        return k // 2, col0 + (k % 2) * (_NW * _CH)

    def idx_copy(k):
        f, c = chunk(k)
        return pltpu.async_copy(
            idx_hbm.at[f, pl.ds(c, _CH)], idx_v.at[k % _NB], isem)

    icopies[0] = idx_copy(0)
    icopies[1] = idx_copy(1)
    icopies[0].wait()
    gathers[0] = pltpu.async_copy(table_hbm.at[idx_v.at[0]], rows[0], gsem)
    for k in range(_NCHUNK):
        cur = k % _NB
        if k + 2 < _NCHUNK:
            icopies[(k + 2) % _NB] = idx_copy(k + 2)
        if k + 1 < _NCHUNK:
            nxt = (k + 1) % _NB
            if writes[nxt] is not None:
                writes[nxt].wait()
            icopies[nxt].wait()
            gathers[nxt] = pltpu.async_copy(
                table_hbm.at[idx_v.at[nxt]], rows[nxt], gsem)
        gathers[cur].wait()
        f, c = chunk(k)
        writes[cur] = pltpu.async_copy(
            rows[cur], out_hbm.at[f, pl.ds(c, _CH)], osem)
    writes[(_NCHUNK - 1) % _NB].wait()
    writes[(_NCHUNK - 2) % _NB].wait()
    writes[(_NCHUNK - 3) % _NB].wait()


def kernel(idx, table):
    idx_t = jnp.swapaxes(idx, 0, 1).astype(jnp.int32)
    t_pad = jnp.pad(table, ((0, 0), (0, _PADW - _DIM)))
    out_t = _gather(t_pad, idx_t)
    return jnp.swapaxes(out_t[:, :, :_DIM], 0, 1)
